# baseline (speedup 1.0000x reference)
import functools

import jax
import jax.numpy as jnp
from jax import lax
from jax.experimental import pallas as pl
from jax.experimental.pallas import tpu as pltpu
from jax.experimental.pallas import tpu_sc as plsc

EMB = 64
HID = 128
SEQ = 200
BATCH = 4096
TAGS = 32

NC = 2
NS = 16
NW = NC * NS
TOTAL = BATCH * SEQ
PER_W = TOTAL // NW
GCHUNK = 128
NCHUNK = PER_W // GCHUNK


def _sc_gather_body(emb_hbm, idx_hbm, out_hbm, idx_v, rows_v, gsem):
    wid = lax.axis_index("s") * NC + lax.axis_index("c")
    row0 = wid * NCHUNK
    pltpu.sync_copy(idx_hbm.at[pl.ds(row0, NCHUNK)], idx_v)

    def step(g, carry):
        pltpu.async_copy(emb_hbm.at[idx_v.at[g]], rows_v, gsem).wait()
        pltpu.sync_copy(rows_v, out_hbm.at[pl.ds((row0 + g) * GCHUNK, GCHUNK)])
        return carry

    lax.fori_loop(0, NCHUNK, step, 0)


def _make_sc_gather():
    mesh = plsc.VectorSubcoreMesh(core_axis_name="c", subcore_axis_name="s")
    return pl.kernel(
        _sc_gather_body,
        out_type=jax.ShapeDtypeStruct((TOTAL, EMB), jnp.float32),
        mesh=mesh,
        compiler_params=pltpu.CompilerParams(use_tc_tiling_on_sc=False),
        scratch_types=[
            pltpu.VMEM((NCHUNK, GCHUNK), jnp.int32),
            pltpu.VMEM((GCHUNK, EMB), jnp.float32),
            pltpu.SemaphoreType.DMA,
        ],
    )


CB = 16


def _mlp_body(rows_ref, w1_ref, b1_ref, w2_ref, b2_ref, out_ref):
    rows = rows_ref[...]
    h = lax.dot_general(rows, w1_ref[...], (((1,), (1,)), ((), ())),
                        preferred_element_type=jnp.float32)
    h = jnp.maximum(h + b1_ref[...], 0.0)
    m = jnp.max(h.reshape(CB, SEQ, HID), axis=1)
    out_ref[...] = lax.dot_general(m, w2_ref[...], (((1,), (1,)), ((), ())),
                                   preferred_element_type=jnp.float32) + b2_ref[...]


def _make_mlp(interpret=False):
    return pl.pallas_call(
        _mlp_body,
        grid=(BATCH // CB,),
        in_specs=[
            pl.BlockSpec((CB * SEQ, EMB), lambda i: (i, 0)),
            pl.BlockSpec((HID, EMB), lambda i: (0, 0)),
            pl.BlockSpec((1, HID), lambda i: (0, 0)),
            pl.BlockSpec((TAGS, HID), lambda i: (0, 0)),
            pl.BlockSpec((1, TAGS), lambda i: (0, 0)),
        ],
        out_specs=pl.BlockSpec((CB, TAGS), lambda i: (i, 0)),
        out_shape=jax.ShapeDtypeStruct((BATCH, TAGS), jnp.float32),
        interpret=interpret,
    )


def kernel(x, emb, W1, b1, W2, b2):
    idx = x.reshape(TOTAL // GCHUNK, GCHUNK).astype(jnp.int32)
    rows = _make_sc_gather()(emb, idx)
    return _make_mlp()(rows, W1, b1.reshape(1, HID), W2, b2.reshape(1, TAGS))


# speedup vs baseline: 1.6900x; 1.6900x over previous
import functools

import jax
import jax.numpy as jnp
from jax import lax
from jax.experimental import pallas as pl
from jax.experimental.pallas import tpu as pltpu
from jax.experimental.pallas import tpu_sc as plsc

VOCAB = 1000000
EMB = 64
HID = 128
SEQ = 200
BATCH = 4096
TAGS = 32

NC = 2
NS = 16
NW = NC * NS
PER_W = BATCH * SEQ // NW
EPW = BATCH // NW
NBUF = 2
NLANE = 8


TBR = 10000


def _tbuild_body(emb_ref, w1_ref, t_ref):
    t_ref[...] = lax.dot_general(
        emb_ref[...], w1_ref[...], (((1,), (1,)), ((), ())),
        preferred_element_type=jnp.float32)


def _make_tbuild():
    return pl.pallas_call(
        _tbuild_body,
        grid=(VOCAB // TBR,),
        in_specs=[
            pl.BlockSpec((TBR, EMB), lambda i: (i, 0)),
            pl.BlockSpec((HID, EMB), lambda i: (0, 0)),
        ],
        out_specs=pl.BlockSpec((TBR, HID), lambda i: (i, 0)),
        out_shape=jax.ShapeDtypeStruct((VOCAB, HID), jnp.float32),
    )


MROWS = 16


def _sc_gathermax_body(t_hbm, idx_hbm, out_hbm, idx_v, rows_v, mbuf, gsems):
    wid = lax.axis_index("s") * NC + lax.axis_index("c")
    tok0 = pl.multiple_of(wid * PER_W, 8)
    e0 = pl.multiple_of(wid * EPW, 8)
    pltpu.sync_copy(idx_hbm.at[pl.ds(tok0, PER_W)], idx_v)

    def gdescr(e, b):
        off = pl.multiple_of(e * SEQ, 8)
        c1 = pltpu.make_async_copy(
            t_hbm.at[idx_v.at[pl.ds(off, 128)]],
            rows_v.at[b, pl.ds(0, 128)], gsems.at[b])
        c2 = pltpu.make_async_copy(
            t_hbm.at[idx_v.at[pl.ds(off + 128, SEQ - 128)]],
            rows_v.at[b, pl.ds(128, SEQ - 128)], gsems.at[b])
        return c1, c2

    def gstart(e, b):
        c1, c2 = gdescr(e, b)
        c1.start()
        c2.start()

    def gwait(e, b):
        c1, c2 = gdescr(e, b)
        c1.wait()
        c2.wait()

    def emax(b):
        def body(r, acc):
            return tuple(
                jnp.maximum(acc[k], rows_v[b, r, 16 * k:16 * (k + 1)])
                for k in range(NLANE))
        acc0 = tuple(
            jnp.full((16,), -jnp.inf, jnp.float32) for _ in range(NLANE))
        return lax.fori_loop(0, SEQ, body, acc0)

    for b in range(NBUF):
        gstart(b, b)

    def group(g, carry):
        for b in range(NBUF):
            e = g * NBUF + b
            gwait(e, b)
            acc = emax(b)
            m = lax.rem(e, MROWS)
            for k in range(NLANE):
                mbuf[m, 16 * k:16 * (k + 1)] = acc[k]

            @pl.when(e + NBUF < EPW)
            def _():
                gstart(e + NBUF, b)

            @pl.when(lax.rem(e, MROWS) == MROWS - 1)
            def _():
                row = pl.multiple_of(e0 + e - (MROWS - 1), 8)
                pltpu.sync_copy(mbuf, out_hbm.at[pl.ds(row, MROWS)])

        return carry

    lax.fori_loop(0, EPW // NBUF, group, 0)


def _make_sc_gathermax():
    mesh = plsc.VectorSubcoreMesh(core_axis_name="c", subcore_axis_name="s")
    return pl.kernel(
        _sc_gathermax_body,
        out_type=jax.ShapeDtypeStruct((BATCH, HID), jnp.float32),
        mesh=mesh,
        scratch_types=[
            pltpu.VMEM((PER_W,), jnp.int32),
            pltpu.VMEM((NBUF, SEQ, HID), jnp.float32),
            pltpu.VMEM((MROWS, HID), jnp.float32),
            pltpu.SemaphoreType.DMA((NBUF,)),
        ],
    )


def _head_body(m_ref, b1_ref, w2_ref, b2_ref, out_ref):
    h = jnp.maximum(m_ref[...] + b1_ref[...], 0.0)
    out_ref[...] = lax.dot_general(
        h, w2_ref[...], (((1,), (1,)), ((), ())),
        preferred_element_type=jnp.float32) + b2_ref[...]


def _make_head(interpret=False):
    return pl.pallas_call(
        _head_body,
        grid=(1,),
        in_specs=[
            pl.BlockSpec((BATCH, HID), lambda i: (0, 0)),
            pl.BlockSpec((1, HID), lambda i: (0, 0)),
            pl.BlockSpec((TAGS, HID), lambda i: (0, 0)),
            pl.BlockSpec((1, TAGS), lambda i: (0, 0)),
        ],
        out_specs=pl.BlockSpec((BATCH, TAGS), lambda i: (0, 0)),
        out_shape=jax.ShapeDtypeStruct((BATCH, TAGS), jnp.float32),
        interpret=interpret,
    )


def kernel(x, emb, W1, b1, W2, b2):
    t = _make_tbuild()(emb, W1)
    idx = x.reshape(-1).astype(jnp.int32)
    m = _make_sc_gathermax()(t, idx)
    return _make_head()(m, b1.reshape(1, HID), W2, b2.reshape(1, TAGS))


# speedup vs baseline: 1.7838x; 1.0555x over previous
import functools

import jax
import jax.numpy as jnp
from jax import lax
from jax.experimental import pallas as pl
from jax.experimental.pallas import tpu as pltpu
from jax.experimental.pallas import tpu_sc as plsc

VOCAB = 1000000
EMB = 64
HID = 128
SEQ = 200
BATCH = 4096
TAGS = 32

NC = 2
NS = 16
NW = NC * NS
PER_W = BATCH * SEQ // NW
EPW = BATCH // NW
NBUF = 4
NB16 = 4
HWORDS = HID // 2


TBR = 16384


def _tbuild_body(embt_ref, w1_ref, t_ref):
    t = lax.dot_general(
        embt_ref[...], w1_ref[...], (((0,), (1,)), ((), ())),
        preferred_element_type=jnp.float32)
    t_ref[...] = pltpu.bitcast(t.astype(jnp.bfloat16), jnp.int32)


def _make_tbuild():
    return pl.pallas_call(
        _tbuild_body,
        grid=((VOCAB + TBR - 1) // TBR,),
        in_specs=[
            pl.BlockSpec((EMB, TBR), lambda i: (0, i)),
            pl.BlockSpec((HID, EMB), lambda i: (0, 0)),
        ],
        out_specs=pl.BlockSpec((TBR // 2, HID), lambda i: (i, 0)),
        out_shape=jax.ShapeDtypeStruct((VOCAB // 2, HID), jnp.int32),
    )


MROWS = 8
NEG = -3.0e38


def _sc_gathermax_body(t_hbm, idx_hbm, out_hbm, idx_v, pairs_v, pars_v,
                       rows_v, mbuf, gsems):
    wid = lax.axis_index("s") * NC + lax.axis_index("c")
    tok0 = pl.multiple_of(wid * PER_W, 8)
    e0 = pl.multiple_of(wid * EPW, 8)
    SLAB = PER_W // 2

    def load_slab(which):
        pltpu.sync_copy(idx_hbm.at[pl.ds(tok0 + which * SLAB, SLAB)],
                        idx_v.at[pl.ds(0, SLAB)])

    load_slab(0)

    def gdescr(e, b):
        c1 = pltpu.make_async_copy(
            t_hbm.at[pairs_v.at[b, pl.ds(0, 128)]],
            rows_v.at[b, pl.ds(0, 128)], gsems.at[b])
        c2 = pltpu.make_async_copy(
            t_hbm.at[pairs_v.at[b, pl.ds(128, SEQ - 128)]],
            rows_v.at[b, pl.ds(128, SEQ - 128)], gsems.at[b])
        return c1, c2

    def gstart(e, b):
        off = pl.multiple_of(lax.rem(e, EPW // 2) * SEQ, 8)
        for i in range(13):
            v = idx_v[pl.ds(off + 16 * i, 16)]
            pairs_v[b, 16 * i:16 * i + 16] = lax.shift_right_logical(v, 1)
            pars_v[b, 16 * i:16 * i + 16] = jnp.bitwise_and(v, 1)
        c1, c2 = gdescr(e, b)
        c1.start()
        c2.start()

    def gwait(e, b):
        c1, c2 = gdescr(e, b)
        c1.wait()
        c2.wait()

    def emax(e, b):
        def block8(g, acc):
            pv = pars_v[b, pl.ds(8 * g, 16)]
            r0 = 8 * g
            for j in range(8):
                p = pv[j] == 1
                new = []
                for k in range(8):
                    w = rows_v[b, r0 + j, 16 * k:16 * (k + 1)]
                    v32 = plsc.bitcast(w, jnp.bfloat16)
                    lo, hi = plsc.unpack(v32, format=plsc.PackFormat.INTERLEAVED)
                    new.append(jnp.maximum(acc[k], jnp.where(p, hi, lo)))
                acc = tuple(new)
            return acc

        acc0 = tuple(
            jnp.full((16,), NEG, jnp.float32) for _ in range(8))
        return lax.fori_loop(0, SEQ // 8, block8, acc0)

    for b in range(NBUF):
        gstart(b, b)

    def group(g, carry):
        for b in range(NBUF):
            e = g * NBUF + b
            gwait(e, b)
            acc = emax(e, b)
            m = lax.rem(e, MROWS)
            for k in range(8):
                mbuf[m, 16 * k:16 * (k + 1)] = acc[k]

            @pl.when(e + NBUF == EPW // 2)
            def _():
                load_slab(1)

            @pl.when(e + NBUF < EPW)
            def _():
                gstart(e + NBUF, b)

            @pl.when(lax.rem(e, MROWS) == MROWS - 1)
            def _():
                row = pl.multiple_of(e0 + e - (MROWS - 1), 8)
                pltpu.sync_copy(mbuf, out_hbm.at[pl.ds(row, MROWS)])

        return carry

    lax.fori_loop(0, EPW // NBUF, group, 0)


def _make_sc_gathermax():
    mesh = plsc.VectorSubcoreMesh(core_axis_name="c", subcore_axis_name="s")
    return pl.kernel(
        _sc_gathermax_body,
        out_type=jax.ShapeDtypeStruct((BATCH, HID), jnp.float32),
        mesh=mesh,
        compiler_params=pltpu.CompilerParams(needs_layout_passes=False),
        scratch_types=[
            pltpu.VMEM((PER_W // 2 + 8,), jnp.int32),
            pltpu.VMEM((NBUF, 256), jnp.int32),
            pltpu.VMEM((NBUF, 256), jnp.int32),
            pltpu.VMEM((NBUF, SEQ, HID), jnp.int32),
            pltpu.VMEM((MROWS, HID), jnp.float32),
            pltpu.SemaphoreType.DMA((NBUF,)),
        ],
    )


def _head_body(m_ref, b1_ref, w2_ref, b2_ref, out_ref):
    h = jnp.maximum(m_ref[...] + b1_ref[...], 0.0)
    out_ref[...] = lax.dot_general(
        h, w2_ref[...], (((1,), (1,)), ((), ())),
        preferred_element_type=jnp.float32) + b2_ref[...]


def _make_head(interpret=False):
    return pl.pallas_call(
        _head_body,
        grid=(1,),
        in_specs=[
            pl.BlockSpec((BATCH, HID), lambda i: (0, 0)),
            pl.BlockSpec((1, HID), lambda i: (0, 0)),
            pl.BlockSpec((TAGS, HID), lambda i: (0, 0)),
            pl.BlockSpec((1, TAGS), lambda i: (0, 0)),
        ],
        out_specs=pl.BlockSpec((BATCH, TAGS), lambda i: (0, 0)),
        out_shape=jax.ShapeDtypeStruct((BATCH, TAGS), jnp.float32),
        interpret=interpret,
    )


def kernel(x, emb, W1, b1, W2, b2):
    t = _make_tbuild()(emb.T, W1)
    idx = x.reshape(-1).astype(jnp.int32)
    m = _make_sc_gathermax()(t, idx)
    return _make_head()(m, b1.reshape(1, HID), W2, b2.reshape(1, TAGS))


# speedup vs baseline: 3.4755x; 1.9484x over previous
import functools

import jax
import jax.numpy as jnp
from jax import lax
from jax.experimental import pallas as pl
from jax.experimental.pallas import tpu as pltpu
from jax.experimental.pallas import tpu_sc as plsc

VOCAB = 1000000
EMB = 64
HID = 128
SEQ = 200
BATCH = 4096
TAGS = 32

NC = 2
NS = 16
NW = NC * NS
PER_W = BATCH * SEQ // NW
EPW = BATCH // NW
NBUF = 4
NLANE = 8


TBR = 16384


def _tbuild_body(embt_ref, w1_ref, t_ref):
    t_ref[...] = lax.dot_general(
        embt_ref[...], w1_ref[...], (((0,), (1,)), ((), ())),
        preferred_element_type=jnp.float32)


def _make_tbuild():
    return pl.pallas_call(
        _tbuild_body,
        grid=((VOCAB + TBR - 1) // TBR,),
        in_specs=[
            pl.BlockSpec((EMB, TBR), lambda i: (0, i)),
            pl.BlockSpec((HID, EMB), lambda i: (0, 0)),
        ],
        out_specs=pl.BlockSpec((TBR, HID), lambda i: (i, 0)),
        out_shape=jax.ShapeDtypeStruct((VOCAB, HID), jnp.float32),
    )


MROWS = 16


def _sc_gathermax_body(t_hbm, idx_hbm, out_hbm, idx_v, rows_v, mbuf, gsems):
    wid = lax.axis_index("s") * NC + lax.axis_index("c")
    tok0 = pl.multiple_of(wid * PER_W, 8)
    e0 = pl.multiple_of(wid * EPW, 8)
    pltpu.sync_copy(idx_hbm.at[pl.ds(tok0, PER_W)], idx_v)

    def gdescr(e, b):
        off = pl.multiple_of(e * SEQ, 8)
        c1 = pltpu.make_async_copy(
            t_hbm.at[idx_v.at[pl.ds(off, 128)]],
            rows_v.at[b, pl.ds(0, 128)], gsems.at[b])
        c2 = pltpu.make_async_copy(
            t_hbm.at[idx_v.at[pl.ds(off + 128, SEQ - 128)]],
            rows_v.at[b, pl.ds(128, SEQ - 128)], gsems.at[b])
        return c1, c2

    def gstart(e, b):
        c1, c2 = gdescr(e, b)
        c1.start()
        c2.start()

    def gwait(e, b):
        c1, c2 = gdescr(e, b)
        c1.wait()
        c2.wait()

    def emax(b):
        def body(r, acc):
            return tuple(
                jnp.maximum(acc[k], rows_v[b, r, 16 * k:16 * (k + 1)])
                for k in range(NLANE))
        acc0 = tuple(
            jnp.full((16,), -jnp.inf, jnp.float32) for _ in range(NLANE))
        return lax.fori_loop(0, SEQ, body, acc0)

    for b in range(NBUF):
        gstart(b, b)

    def group(g, carry):
        for b in range(NBUF):
            e = g * NBUF + b
            gwait(e, b)
            acc = emax(b)
            m = lax.rem(e, MROWS)
            for k in range(NLANE):
                mbuf[m, 16 * k:16 * (k + 1)] = acc[k]

            @pl.when(e + NBUF < EPW)
            def _():
                gstart(e + NBUF, b)

            @pl.when(lax.rem(e, MROWS) == MROWS - 1)
            def _():
                row = pl.multiple_of(e0 + e - (MROWS - 1), 8)
                pltpu.sync_copy(mbuf, out_hbm.at[pl.ds(row, MROWS)])

        return carry

    lax.fori_loop(0, EPW // NBUF, group, 0)


def _make_sc_gathermax():
    mesh = plsc.VectorSubcoreMesh(core_axis_name="c", subcore_axis_name="s")
    return pl.kernel(
        _sc_gathermax_body,
        out_type=jax.ShapeDtypeStruct((BATCH, HID), jnp.float32),
        mesh=mesh,
        scratch_types=[
            pltpu.VMEM((PER_W,), jnp.int32),
            pltpu.VMEM((NBUF, SEQ, HID), jnp.float32),
            pltpu.VMEM((MROWS, HID), jnp.float32),
            pltpu.SemaphoreType.DMA((NBUF,)),
        ],
    )


def _head_body(m_ref, b1_ref, w2_ref, b2_ref, out_ref):
    h = jnp.maximum(m_ref[...] + b1_ref[...], 0.0)
    out_ref[...] = lax.dot_general(
        h, w2_ref[...], (((1,), (1,)), ((), ())),
        preferred_element_type=jnp.float32) + b2_ref[...]


def _make_head(interpret=False):
    return pl.pallas_call(
        _head_body,
        grid=(1,),
        in_specs=[
            pl.BlockSpec((BATCH, HID), lambda i: (0, 0)),
            pl.BlockSpec((1, HID), lambda i: (0, 0)),
            pl.BlockSpec((TAGS, HID), lambda i: (0, 0)),
            pl.BlockSpec((1, TAGS), lambda i: (0, 0)),
        ],
        out_specs=pl.BlockSpec((BATCH, TAGS), lambda i: (0, 0)),
        out_shape=jax.ShapeDtypeStruct((BATCH, TAGS), jnp.float32),
        interpret=interpret,
    )


def kernel(x, emb, W1, b1, W2, b2):
    t = _make_tbuild()(emb.T, W1)
    idx = x.reshape(-1).astype(jnp.int32)
    m = _make_sc_gathermax()(t, idx)
    return _make_head()(m, b1.reshape(1, HID), W2, b2.reshape(1, TAGS))


# speedup vs baseline: 3.5299x; 1.0157x over previous
import functools

import jax
import jax.numpy as jnp
from jax import lax
from jax.experimental import pallas as pl
from jax.experimental.pallas import tpu as pltpu
from jax.experimental.pallas import tpu_sc as plsc

VOCAB = 1000000
EMB = 64
HID = 128
SEQ = 200
BATCH = 4096
TAGS = 32

NC = 2
NS = 16
NW = NC * NS
PER_W = BATCH * SEQ // NW
EPW = BATCH // NW
NBUF = 4
NLANE = 8


TBR = 32768


def _tbuild_body(embt_ref, w1_ref, t_ref):
    t_ref[...] = lax.dot_general(
        embt_ref[...], w1_ref[...], (((0,), (1,)), ((), ())),
        preferred_element_type=jnp.float32)


def _make_tbuild():
    return pl.pallas_call(
        _tbuild_body,
        grid=((VOCAB + TBR - 1) // TBR,),
        in_specs=[
            pl.BlockSpec((EMB, TBR), lambda i: (0, i)),
            pl.BlockSpec((HID, EMB), lambda i: (0, 0)),
        ],
        out_specs=pl.BlockSpec((TBR, HID), lambda i: (i, 0)),
        out_shape=jax.ShapeDtypeStruct((VOCAB, HID), jnp.float32),
    )


MROWS = 16


def _sc_gathermax_body(t_hbm, idx_hbm, out_hbm, idx_v, rows_v, mbuf, gsems):
    wid = lax.axis_index("s") * NC + lax.axis_index("c")
    tok0 = pl.multiple_of(wid * PER_W, 8)
    e0 = pl.multiple_of(wid * EPW, 8)
    pltpu.sync_copy(idx_hbm.at[pl.ds(tok0, PER_W)], idx_v)

    def gdescr(e, b):
        off = pl.multiple_of(e * SEQ, 8)
        c1 = pltpu.make_async_copy(
            t_hbm.at[idx_v.at[pl.ds(off, 128)]],
            rows_v.at[b, pl.ds(0, 128)], gsems.at[b])
        c2 = pltpu.make_async_copy(
            t_hbm.at[idx_v.at[pl.ds(off + 128, SEQ - 128)]],
            rows_v.at[b, pl.ds(128, SEQ - 128)], gsems.at[b])
        return c1, c2

    def gstart(e, b):
        c1, c2 = gdescr(e, b)
        c1.start()
        c2.start()

    def gwait(e, b):
        c1, c2 = gdescr(e, b)
        c1.wait()
        c2.wait()

    def emax(b):
        def body(r, acc):
            return tuple(
                jnp.maximum(acc[k], rows_v[b, r, 16 * k:16 * (k + 1)])
                for k in range(NLANE))
        acc0 = tuple(
            jnp.full((16,), -jnp.inf, jnp.float32) for _ in range(NLANE))
        return lax.fori_loop(0, SEQ, body, acc0)

    for b in range(NBUF):
        gstart(b, b)

    def group(g, carry):
        for b in range(NBUF):
            e = g * NBUF + b
            gwait(e, b)
            acc = emax(b)
            m = lax.rem(e, MROWS)
            for k in range(NLANE):
                mbuf[m, 16 * k:16 * (k + 1)] = acc[k]

            @pl.when(e + NBUF < EPW)
            def _():
                gstart(e + NBUF, b)

            @pl.when(lax.rem(e, MROWS) == MROWS - 1)
            def _():
                row = pl.multiple_of(e0 + e - (MROWS - 1), 8)
                pltpu.sync_copy(mbuf, out_hbm.at[pl.ds(row, MROWS)])

        return carry

    lax.fori_loop(0, EPW // NBUF, group, 0)


def _make_sc_gathermax():
    mesh = plsc.VectorSubcoreMesh(core_axis_name="c", subcore_axis_name="s")
    return pl.kernel(
        _sc_gathermax_body,
        out_type=jax.ShapeDtypeStruct((BATCH, HID), jnp.float32),
        mesh=mesh,
        scratch_types=[
            pltpu.VMEM((PER_W,), jnp.int32),
            pltpu.VMEM((NBUF, SEQ, HID), jnp.float32),
            pltpu.VMEM((MROWS, HID), jnp.float32),
            pltpu.SemaphoreType.DMA((NBUF,)),
        ],
    )


def _head_body(m_ref, b1_ref, w2_ref, b2_ref, out_ref):
    h = jnp.maximum(m_ref[...] + b1_ref[...], 0.0)
    out_ref[...] = lax.dot_general(
        h, w2_ref[...], (((1,), (1,)), ((), ())),
        preferred_element_type=jnp.float32) + b2_ref[...]


def _make_head(interpret=False):
    return pl.pallas_call(
        _head_body,
        grid=(1,),
        in_specs=[
            pl.BlockSpec((BATCH, HID), lambda i: (0, 0)),
            pl.BlockSpec((1, HID), lambda i: (0, 0)),
            pl.BlockSpec((TAGS, HID), lambda i: (0, 0)),
            pl.BlockSpec((1, TAGS), lambda i: (0, 0)),
        ],
        out_specs=pl.BlockSpec((BATCH, TAGS), lambda i: (0, 0)),
        out_shape=jax.ShapeDtypeStruct((BATCH, TAGS), jnp.float32),
        interpret=interpret,
    )


def kernel(x, emb, W1, b1, W2, b2):
    t = _make_tbuild()(emb.T, W1)
    idx = x.reshape(-1).astype(jnp.int32)
    m = _make_sc_gathermax()(t, idx)
    return _make_head()(m, b1.reshape(1, HID), W2, b2.reshape(1, TAGS))
